# baseline (speedup 1.0000x reference)
import functools

import jax
import jax.numpy as jnp
from jax import lax
from jax.experimental import pallas as pl
from jax.experimental.pallas import tpu as pltpu
from jax.experimental.pallas import tpu_sc as plsc

D = 64
K = 5
NC = 2
NS = 16
NW = NC * NS
LANES = 16
PAD = 1.0e9


def _sc_dots(v_i, v_j, neg_t, nodes, ctx):
    B = v_i.shape[0]
    PB = B // NW
    C = min(128, PB)
    NCHUNK = PB // C

    mesh = plsc.VectorSubcoreMesh(core_axis_name="c", subcore_axis_name="s")

    @functools.partial(
        pl.kernel,
        mesh=mesh,
        compiler_params=pltpu.CompilerParams(use_tc_tiling_on_sc=False),
        out_type=jax.ShapeDtypeStruct((B, LANES), jnp.float32),
        scratch_types=[
            pltpu.VMEM((C,), jnp.int32),
            pltpu.VMEM((C,), jnp.int32),
            pltpu.VMEM((K * C,), jnp.int32),
            pltpu.VMEM((C, D), jnp.float32),
            pltpu.VMEM((C, D), jnp.float32),
            pltpu.VMEM((K * C, D), jnp.float32),
            pltpu.VMEM((C, LANES), jnp.float32),
            pltpu.SemaphoreType.DMA,
        ],
    )
    def body(vi_hbm, vj_hbm, negt_hbm, nodes_hbm, ctx_hbm, out_hbm,
             vi_idx, vj_idx, neg_idx, vi_rows, vj_rows, neg_rows, dots, sem):
        wid = lax.axis_index("s") * NC + lax.axis_index("c")
        lane = lax.iota(jnp.int32, 16)
        pad_vec = jnp.where(lane < 1 + K, jnp.float32(0), jnp.float32(PAD))

        def chunk_body(ci, carry):
            base = wid * PB + ci * C
            pltpu.sync_copy(vi_hbm.at[pl.ds(base, C)], vi_idx)
            pltpu.sync_copy(vj_hbm.at[pl.ds(base, C)], vj_idx)
            for k in range(K):
                pltpu.sync_copy(negt_hbm.at[pl.ds(k * B + base, C)],
                                neg_idx.at[pl.ds(k * C, C)])
            copies = [
                pltpu.async_copy(nodes_hbm.at[vi_idx], vi_rows, sem),
                pltpu.async_copy(ctx_hbm.at[vj_idx], vj_rows, sem),
            ]
            for k in range(K):
                copies.append(
                    pltpu.async_copy(ctx_hbm.at[neg_idx.at[pl.ds(k * C, C)]],
                                     neg_rows.at[pl.ds(k * C, C)], sem))
            for c in copies:
                c.wait()

            def lane_sum(x):
                for sh in (8, 4, 2, 1):
                    x = x + x.at[lane ^ sh].get(mode="promise_in_bounds")
                return x

            def elem_body(i, carry2):
                vi_g = [vi_rows[i, pl.ds(g * LANES, LANES)] for g in range(D // LANES)]
                acc = vi_g[0] * vj_rows[i, pl.ds(0, LANES)]
                for g in range(1, D // LANES):
                    acc += vi_g[g] * vj_rows[i, pl.ds(g * LANES, LANES)]
                dvec = jnp.where(lane == 0, lane_sum(acc), pad_vec)
                for k in range(K):
                    nacc = vi_g[0] * neg_rows[k * C + i, pl.ds(0, LANES)]
                    for g in range(1, D // LANES):
                        nacc += vi_g[g] * neg_rows[k * C + i, pl.ds(g * LANES, LANES)]
                    dvec = jnp.where(lane == k + 1, -lane_sum(nacc), dvec)
                dots[i, :] = dvec
                return carry2

            lax.fori_loop(0, C, elem_body, 0)
            pltpu.sync_copy(dots, out_hbm.at[pl.ds(base, C)])
            return carry

        lax.fori_loop(0, NCHUNK, chunk_body, 0)

    return body(v_i, v_j, neg_t, nodes, ctx)


def _tc_loss(dots2d, batch):
    def body(x_ref, o_ref):
        x = x_ref[...]
        ls = jnp.minimum(x, 0.0) - jnp.log1p(jnp.exp(-jnp.abs(x)))
        o_ref[0, 0] = -jnp.sum(ls) / batch

    return pl.pallas_call(
        body,
        out_shape=jax.ShapeDtypeStruct((1, 1), jnp.float32),
        out_specs=pl.BlockSpec(memory_space=pltpu.SMEM),
    )(dots2d)


def kernel(v_i, v_j, negsamples, device, nodes_embeddings, contextnodes_embeddings):
    B = v_i.shape[0]
    vi = v_i.astype(jnp.int32)
    vj = v_j.astype(jnp.int32)
    neg_t = negsamples.astype(jnp.int32).T.reshape(-1)
    dots = _sc_dots(vi, vj, neg_t, nodes_embeddings, contextnodes_embeddings)
    loss = _tc_loss(dots.reshape(B * LANES // 128, 128), B)
    return loss[0, 0]


# speedup vs baseline: 1.0523x; 1.0523x over previous
import functools

import jax
import jax.numpy as jnp
from jax import lax
from jax.experimental import pallas as pl
from jax.experimental.pallas import tpu as pltpu
from jax.experimental.pallas import tpu_sc as plsc

D = 64
DP = 128
K = 5
NC = 2
NS = 16
NW = NC * NS
LANES = 16
PAD = 1.0e9


def _sc_dots(v_i, v_j, neg_t, nodes_p, ctx_p):
    B = v_i.shape[0]
    PB = B // NW
    C = min(128, PB)
    NCHUNK = PB // C
    CR = C // 8

    mesh = plsc.VectorSubcoreMesh(core_axis_name="c", subcore_axis_name="s")

    @functools.partial(
        pl.kernel,
        mesh=mesh,
        out_type=jax.ShapeDtypeStruct((B // 8, 128), jnp.float32),
        scratch_types=[
            pltpu.VMEM((C,), jnp.int32),
            pltpu.VMEM((C,), jnp.int32),
            pltpu.VMEM((K * C,), jnp.int32),
            pltpu.VMEM((C, DP), jnp.float32),
            pltpu.VMEM((C, DP), jnp.float32),
            pltpu.VMEM((K * C, DP), jnp.float32),
            pltpu.VMEM((CR, 128), jnp.float32),
            pltpu.SemaphoreType.DMA,
        ],
    )
    def body(vi_hbm, vj_hbm, negt_hbm, nodes_hbm, ctx_hbm, out_hbm,
             vi_idx, vj_idx, neg_idx, vi_rows, vj_rows, neg_rows, dots, sem):
        wid = lax.axis_index("s") * NC + lax.axis_index("c")
        lane = lax.iota(jnp.int32, 16)
        pad_vec = jnp.where(lane < 1 + K, jnp.float32(0), jnp.float32(PAD))

        def chunk_body(ci, carry):
            base = wid * PB + ci * C
            pltpu.sync_copy(vi_hbm.at[pl.ds(base, C)], vi_idx)
            pltpu.sync_copy(vj_hbm.at[pl.ds(base, C)], vj_idx)
            for k in range(K):
                pltpu.sync_copy(negt_hbm.at[pl.ds(k * B + base, C)],
                                neg_idx.at[pl.ds(k * C, C)])
            copies = [
                pltpu.async_copy(nodes_hbm.at[vi_idx], vi_rows, sem),
                pltpu.async_copy(ctx_hbm.at[vj_idx], vj_rows, sem),
            ]
            for k in range(K):
                copies.append(
                    pltpu.async_copy(ctx_hbm.at[neg_idx.at[pl.ds(k * C, C)]],
                                     neg_rows.at[pl.ds(k * C, C)], sem))
            for c in copies:
                c.wait()

            def lane_sum(x):
                for sh in (8, 4, 2, 1):
                    x = x + x.at[lane ^ sh].get(mode="promise_in_bounds")
                return x

            def elem_body(i, carry2):
                vi_g = [vi_rows[i, pl.ds(g * LANES, LANES)] for g in range(D // LANES)]
                acc = vi_g[0] * vj_rows[i, pl.ds(0, LANES)]
                for g in range(1, D // LANES):
                    acc += vi_g[g] * vj_rows[i, pl.ds(g * LANES, LANES)]
                dvec = jnp.where(lane == 0, lane_sum(acc), pad_vec)
                for k in range(K):
                    nacc = vi_g[0] * neg_rows[k * C + i, pl.ds(0, LANES)]
                    for g in range(1, D // LANES):
                        nacc += vi_g[g] * neg_rows[k * C + i, pl.ds(g * LANES, LANES)]
                    dvec = jnp.where(lane == k + 1, -lane_sum(nacc), dvec)
                dots[i // 8, pl.ds((i % 8) * LANES, LANES)] = dvec
                return carry2

            lax.fori_loop(0, C, elem_body, 0)
            pltpu.sync_copy(
                dots, out_hbm.at[pl.ds(pl.multiple_of(base // 8, 8), CR)])
            return carry

        lax.fori_loop(0, NCHUNK, chunk_body, 0)

    return body(v_i, v_j, neg_t, nodes_p, ctx_p)


def _tc_loss(dots2d, batch):
    def body(x_ref, o_ref):
        x = x_ref[...]
        ls = jnp.minimum(x, 0.0) - jnp.log1p(jnp.exp(-jnp.abs(x)))
        o_ref[0, 0] = -jnp.sum(ls) / batch

    return pl.pallas_call(
        body,
        out_shape=jax.ShapeDtypeStruct((1, 1), jnp.float32),
        out_specs=pl.BlockSpec(memory_space=pltpu.SMEM),
    )(dots2d)


def kernel(v_i, v_j, negsamples, device, nodes_embeddings, contextnodes_embeddings):
    B = v_i.shape[0]
    vi = v_i.astype(jnp.int32)
    vj = v_j.astype(jnp.int32)
    neg_t = negsamples.astype(jnp.int32).T.reshape(-1)
    nodes_p = jnp.pad(nodes_embeddings, ((0, 0), (0, DP - D)))
    ctx_p = jnp.pad(contextnodes_embeddings, ((0, 0), (0, DP - D)))
    dots = _sc_dots(vi, vj, neg_t, nodes_p, ctx_p)
    loss = _tc_loss(dots, B)
    return loss[0, 0]
